# baseline (speedup 1.0000x reference)
import functools
import math

import jax
import jax.numpy as jnp
import numpy as np
from jax import lax
from jax.experimental import pallas as pl
from jax.experimental.pallas import tpu as pltpu
from jax.experimental.pallas import tpu_sc as plsc

H = 4
F = 64
D = 16
NET = 5
NLAYER = 2
E = 160000
N_COMPANY = 50000
NHALF = 16672
NPASS = 3
ETOT = NET * E
CHUNK = 64
NCHUNK = 12512
ETOTP = NCHUNK * CHUNK
NC = 2
NS = 16
TRIPS = NCHUNK // NS
PAIRS = TRIPS // 2
ACC_DATA_ROWS = NHALF // 4
ACC_FLUSH = 264
ACC_ROWS = NS * ACC_FLUSH
DEN_ROWS = 264


def _edge_body(lo, kv_hbm, q_hbm, sg_hbm, dg_hbm, zacc_hbm, zden_hbm,
               acc_out, den_out,
               acc_sp, den_sp,
               sgv0, sgv1, dgv0, dgv1, kv0, kv1, q0b, q1b,
               accidx0, accidx1, denrv0, denrv1,
               col4v, prev4v, dencv, prevdv, msg, den_msg,
               gsem0, gsem1, ssem):
    c = lax.axis_index("c")
    s = lax.axis_index("s")

    sgv = (sgv0, sgv1)
    dgv = (dgv0, dgv1)
    kvb = (kv0, kv1)
    qb_ = (q0b, q1b)
    accidx = (accidx0, accidx1)
    denrv = (denrv0, denrv1)
    gsem = (gsem0, gsem1)

    pltpu.sync_copy(zacc_hbm, acc_sp.at[pl.ds(s * ACC_FLUSH, ACC_FLUSH)])

    @pl.when(s == 0)
    def _():
        pltpu.sync_copy(zden_hbm, den_sp)

    zero16 = jnp.zeros((16,), jnp.float32)
    zero16i = jnp.zeros((16,), jnp.int32)

    def zero_body(e, carry):
        for g in range(8):
            msg[e, pl.ds(g * 16, 16)] = zero16
        return carry

    lax.fori_loop(0, CHUNK, zero_body, 0)

    def zero_dben(e, carry):
        for g in range(8):
            den_msg[e, pl.ds(g * 16, 16)] = zero16
        return carry

    lax.fori_loop(0, CHUNK + 1, zero_dben, 0)

    def zero_idx_body(g, carry):
        prev4v[pl.ds(g * 16, 16)] = zero16i
        prevdv[pl.ds(g * 16, 16)] = zero16i
        return carry

    lax.fori_loop(0, (CHUNK + 16) // 16, zero_idx_body, 0)
    plsc.subcore_barrier()

    lane = lax.iota(jnp.int32, 16)
    perm = [lane ^ 8, lane ^ 4, lane ^ 2, lane ^ 1]

    def _sum_splat(x):
        for p in perm:
            x = x + x.at[p].get(mode="promise_in_bounds")
        return x

    cb = c * 64
    qoff = c * 32

    def load_and_fire(b, j):
        pltpu.sync_copy(sg_hbm.at[pl.ds(j * CHUNK, CHUNK)], sgv[b])
        pltpu.sync_copy(dg_hbm.at[pl.ds(j * CHUNK, CHUNK)],
                        dgv[b].at[pl.ds(0, CHUNK)])
        pltpu.async_copy(kv_hbm.at[sgv[b]], kvb[b], gsem[b])
        pltpu.async_copy(q_hbm.at[dgv[b].at[pl.ds(0, CHUNK)]], qb_[b],
                         gsem[b])

    for b in range(2):
        load_and_fire(b, s + b * NS)

    def run_chunk(tp, b):
        t = 2 * tp + b
        o = 1 - b

        @pl.when(t >= 1)
        def _():
            pltpu.make_async_copy(msg, acc_sp.at[accidx[o]], ssem).wait()
            pltpu.make_async_copy(den_msg.at[pl.ds(0, CHUNK)],
                                  den_sp.at[denrv[o]], ssem).wait()

        pltpu.make_async_copy(kv_hbm.at[sgv[b]], kvb[b], gsem[b]).wait()
        pltpu.make_async_copy(q_hbm.at[dgv[b].at[pl.ds(0, CHUNK)]],
                              qb_[b], gsem[b]).wait()

        def idx_body(g, carry2):
            d16 = dgv[b][pl.ds(g * 16, 16)]
            dl = d16 - lo
            inh = (dl >= 0) & (dl < NHALF)
            accidx[b][pl.ds(g * 16, 16)] = jnp.where(
                inh, dl >> 2, ACC_DATA_ROWS)
            col4v[pl.ds(g * 16, 16)] = (dl & 3) * 32
            denrv[b][pl.ds(g * 16, 16)] = jnp.where(
                inh, dl >> 6, DEN_ROWS - 3)
            dencv[pl.ds(g * 16, 16)] = (dl & 63) * 2
            return carry2

        lax.fori_loop(0, CHUNK // 16, idx_body, 0)

        def edge_body(e, carry2):
            prev4 = prev4v[pl.ds(e, 16)][0]
            msg[e, pl.ds(prev4, 16)] = zero16
            msg[e, pl.ds(prev4 + 16, 16)] = zero16
            prevd = prevdv[pl.ds(e, 16)][0]
            den_msg[e, pl.ds(prevd, 16)] = zero16
            k0 = kvb[b][e, pl.ds(cb, 16)]
            k1 = kvb[b][e, pl.ds(cb + 16, 16)]
            v0 = kvb[b][e, pl.ds(cb + 32, 16)]
            v1 = kvb[b][e, pl.ds(cb + 48, 16)]
            q0 = qb_[b][e, pl.ds(qoff, 16)]
            q1 = qb_[b][e, pl.ds(qoff + 16, 16)]
            e0 = jnp.exp(_sum_splat(k0 * q0))
            e1 = jnp.exp(_sum_splat(k1 * q1))
            col4 = col4v[pl.ds(e, 16)][0]
            msg[e, pl.ds(col4, 16)] = v0 * e0
            msg[e, pl.ds(col4 + 16, 16)] = v1 * e1
            exd = jnp.where(lane == 0, e0, jnp.where(lane == 1, e1, 0.0))
            den_msg[e, pl.ds(dencv[pl.ds(e, 16)][0], 16)] = exd
            return carry2

        lax.fori_loop(0, CHUNK, edge_body, 0)

        def save_body(g, carry2):
            prev4v[pl.ds(g * 16, 16)] = col4v[pl.ds(g * 16, 16)]
            prevdv[pl.ds(g * 16, 16)] = dencv[pl.ds(g * 16, 16)]
            return carry2

        lax.fori_loop(0, CHUNK // 16, save_body, 0)

        pltpu.async_copy(msg, acc_sp.at[accidx[b]], ssem, add=True)
        pltpu.async_copy(den_msg.at[pl.ds(0, CHUNK)],
                         den_sp.at[denrv[b]], ssem, add=True)

        @pl.when(tp < PAIRS - 1)
        def _():
            load_and_fire(b, s + (t + 2) * NS)

    def pair_body(tp, carry):
        run_chunk(tp, 0)
        run_chunk(tp, 1)
        return carry

    lax.fori_loop(0, PAIRS, pair_body, 0)

    pltpu.make_async_copy(msg, acc_sp.at[accidx[1]], ssem).wait()
    pltpu.make_async_copy(den_msg.at[pl.ds(0, CHUNK)],
                          den_sp.at[denrv[1]], ssem).wait()
    plsc.subcore_barrier()

    pltpu.sync_copy(acc_sp.at[pl.ds(s * ACC_FLUSH, ACC_FLUSH)],
                    acc_out.at[c, s])

    @pl.when(s == 0)
    def _():
        pltpu.sync_copy(den_sp, den_out.at[c])


@functools.partial(jax.jit, static_argnums=0)
def _edge_pass(lo, kv, q2, sg, dg, zacc, zden):
    mesh = plsc.VectorSubcoreMesh(core_axis_name="c", subcore_axis_name="s",
                                  num_cores=NC, num_subcores=NS)
    return pl.kernel(
        functools.partial(_edge_body, lo),
        out_type=(
            jax.ShapeDtypeStruct((NC, NS, ACC_FLUSH, 128), jnp.float32),
            jax.ShapeDtypeStruct((NC, DEN_ROWS, 128), jnp.float32),
        ),
        mesh=mesh,
        scratch_types=[
            pltpu.VMEM_SHARED((ACC_ROWS, 128), jnp.float32),
            pltpu.VMEM_SHARED((DEN_ROWS, 128), jnp.float32),
            pltpu.VMEM((CHUNK,), jnp.int32),
            pltpu.VMEM((CHUNK,), jnp.int32),
            pltpu.VMEM((CHUNK + 16,), jnp.int32),
            pltpu.VMEM((CHUNK + 16,), jnp.int32),
            pltpu.VMEM((CHUNK, 128), jnp.float32),
            pltpu.VMEM((CHUNK, 128), jnp.float32),
            pltpu.VMEM((CHUNK, 128), jnp.float32),
            pltpu.VMEM((CHUNK, 128), jnp.float32),
            pltpu.VMEM((CHUNK,), jnp.int32),
            pltpu.VMEM((CHUNK,), jnp.int32),
            pltpu.VMEM((CHUNK,), jnp.int32),
            pltpu.VMEM((CHUNK,), jnp.int32),
            pltpu.VMEM((CHUNK + 16,), jnp.int32),
            pltpu.VMEM((CHUNK + 16,), jnp.int32),
            pltpu.VMEM((CHUNK + 16,), jnp.int32),
            pltpu.VMEM((CHUNK + 16,), jnp.int32),
            pltpu.VMEM((CHUNK, 128), jnp.float32),
            pltpu.VMEM((CHUNK + 1, 128), jnp.float32),
            pltpu.SemaphoreType.DMA,
            pltpu.SemaphoreType.DMA,
            pltpu.SemaphoreType.DMA,
        ],
        compiler_params=pltpu.CompilerParams(needs_layout_passes=False),
    )(kv, q2, sg, dg, zacc, zden)


def kernel(x_company, x_offshore_entity, x_person, edge_index_owns,
           edge_index_controls, edge_index_alias, edge_index_phoenix_successor,
           edge_index_issued_invoice_to, Win, b_in, Wkqv, b_kqv, Wk_rel,
           Wv_rel, p_rel, Wout, b_out, skip, Wc1, b_c1, Wc2, b_c2):
    NT = ['company', 'offshore_entity', 'person']
    xs = [x_company, x_offshore_entity, x_person]
    x = {}
    for i, nt in enumerate(NT):
        x[nt] = xs[i] @ Win[i] + b_in[i]

    ei = [edge_index_owns, edge_index_controls, edge_index_alias,
          edge_index_phoenix_successor, edge_index_issued_invoice_to]
    src_of = ['company', 'person', 'company', 'company', 'company']
    src_off = [0, 50000, 100000, 150000, 200000]
    npad = ETOTP - ETOT
    sg = jnp.concatenate(
        [ei[j][0] + src_off[j] for j in range(NET)]
        + [jnp.zeros((npad,), jnp.int32)])
    dg = jnp.concatenate([ei[j][1] for j in range(NET)]
                         + [jnp.full((npad,), N_COMPANY, jnp.int32)])
    zacc = jnp.zeros((ACC_FLUSH, 128), jnp.float32)
    zden = jnp.zeros((DEN_ROWS, 128), jnp.float32)

    inv_sqrt_d = 1.0 / math.sqrt(D)
    for l in range(NLAYER):
        kd, qd, vd = {}, {}, {}
        for i, nt in enumerate(NT):
            kqv = x[nt] @ Wkqv[l, i] + b_kqv[l, i]
            k_, q_, v_ = jnp.split(kqv, 3, axis=1)
            kd[nt] = k_.reshape(-1, H, D)
            qd[nt] = q_.reshape(-1, H, D)
            vd[nt] = v_.reshape(-1, H, D)
        q2 = jnp.pad(qd['company'].reshape(-1, F), ((0, 8), (0, 64)))
        ks_l, vs_l = [], []
        for j in range(NET):
            sname = src_of[j]
            idx = np.arange(H) * NET + j
            scale = (p_rel[l, j] * inv_sqrt_d)[None, :, None]
            ks_l.append(jnp.einsum('nhd,hde->nhe', kd[sname],
                                   Wk_rel[l][idx]) * scale)
            vs_l.append(jnp.einsum('nhd,hde->nhe', vd[sname], Wv_rel[l][idx]))
        k_all = jnp.concatenate(ks_l)
        v_all = jnp.concatenate(vs_l)
        kv = jnp.concatenate([
            k_all[:, 0:2].reshape(-1, 32), v_all[:, 0:2].reshape(-1, 32),
            k_all[:, 2:4].reshape(-1, 32), v_all[:, 2:4].reshape(-1, 32),
        ], axis=1)

        accs, dens = [], []
        for h in range(NPASS):
            acc_out, den_out = _edge_pass(h * NHALF, kv, q2, sg, dg,
                                          zacc, zden)
            acc_full = acc_out.reshape(NC, ACC_ROWS, 128)
            accs.append(jnp.concatenate(
                [acc_full[0, :ACC_DATA_ROWS].reshape(-1, 32),
                 acc_full[1, :ACC_DATA_ROWS].reshape(-1, 32)], axis=1))
            dens.append(jnp.concatenate(
                [den_out[0].reshape(-1, 2)[:NHALF],
                 den_out[1].reshape(-1, 2)[:NHALF]], axis=1))
        acc = jnp.concatenate(accs, axis=0)[:N_COMPANY]
        den = jnp.concatenate(dens, axis=0)[:N_COMPANY]
        o = (acc.reshape(-1, H, D) / (den[..., None] + 1e-16)).reshape(-1, F)
        a = jax.nn.gelu(o, approximate=False) @ Wout[l, 0] + b_out[l, 0]
        beta = jax.nn.sigmoid(skip[l, 0])
        x['company'] = jax.nn.elu(beta * a + (1.0 - beta) * x['company'])

    outs = []
    for i, nt in enumerate(NT):
        h1 = jax.nn.relu(x[nt] @ Wc1[i] + b_c1[i])
        outs.append((h1 @ Wc2[i] + b_c2[i])[:, 0])
    return tuple(outs)


# speedup vs baseline: 1.0235x; 1.0235x over previous
import functools
import math

import jax
import jax.numpy as jnp
import numpy as np
from jax import lax
from jax.experimental import pallas as pl
from jax.experimental.pallas import tpu as pltpu
from jax.experimental.pallas import tpu_sc as plsc

H = 4
F = 64
D = 16
NET = 5
NLAYER = 2
E = 160000
N_COMPANY = 50000
NHALF = 16672
NPASS = 3
ETOT = NET * E
CHUNK = 64
NCHUNK = 12512
ETOTP = NCHUNK * CHUNK
NC = 2
NS = 16
TRIPS = NCHUNK // NS
PAIRS = TRIPS // 2
ACC_DATA_ROWS = NHALF // 4
ACC_FLUSH = 264
ACC_ROWS = NS * ACC_FLUSH
DEN_ROWS = 264


def _edge_body(lo, kv_hbm, q_hbm, sg_hbm, dg_hbm, zacc_hbm, zden_hbm,
               acc_out, den_out,
               acc_sp, den_sp,
               sgv0, sgv1, dgv0, dgv1, kv0, kv1, q0b, q1b,
               accidx0, accidx1, denrv0, denrv1,
               col4v, prev4v, dencv, prevdv, msg, den_msg,
               gsem0, gsem1, ssem):
    c = lax.axis_index("c")
    s = lax.axis_index("s")

    sgv = (sgv0, sgv1)
    dgv = (dgv0, dgv1)
    kvb = (kv0, kv1)
    qb_ = (q0b, q1b)
    accidx = (accidx0, accidx1)
    denrv = (denrv0, denrv1)
    gsem = (gsem0, gsem1)

    pltpu.sync_copy(zacc_hbm, acc_sp.at[pl.ds(s * ACC_FLUSH, ACC_FLUSH)])

    @pl.when(s == 0)
    def _():
        pltpu.sync_copy(zden_hbm, den_sp)

    zero16 = jnp.zeros((16,), jnp.float32)
    zero16i = jnp.zeros((16,), jnp.int32)

    def zero_body(e, carry):
        for g in range(8):
            msg[e, pl.ds(g * 16, 16)] = zero16
        return carry

    lax.fori_loop(0, CHUNK, zero_body, 0)

    def zero_dben(e, carry):
        for g in range(8):
            den_msg[e, pl.ds(g * 16, 16)] = zero16
        return carry

    lax.fori_loop(0, CHUNK + 1, zero_dben, 0)

    def zero_idx_body(g, carry):
        prev4v[pl.ds(g * 16, 16)] = zero16i
        prevdv[pl.ds(g * 16, 16)] = zero16i
        return carry

    lax.fori_loop(0, (CHUNK + 16) // 16, zero_idx_body, 0)
    plsc.subcore_barrier()

    lane = lax.iota(jnp.int32, 16)
    perm = [lane ^ 8, lane ^ 4, lane ^ 2, lane ^ 1]

    def _sum_splat(x):
        for p in perm:
            x = x + x.at[p].get(mode="promise_in_bounds")
        return x

    cb = c * 64
    qoff = c * 32

    def load_and_fire(b, j):
        pltpu.sync_copy(sg_hbm.at[pl.ds(j * CHUNK, CHUNK)], sgv[b])
        pltpu.sync_copy(dg_hbm.at[pl.ds(j * CHUNK, CHUNK)],
                        dgv[b].at[pl.ds(0, CHUNK)])
        pltpu.async_copy(kv_hbm.at[sgv[b]], kvb[b], gsem[b])
        pltpu.async_copy(q_hbm.at[dgv[b].at[pl.ds(0, CHUNK)]], qb_[b],
                         gsem[b])

    for b in range(2):
        load_and_fire(b, s + b * NS)

    def run_chunk(tp, b):
        t = 2 * tp + b
        o = 1 - b

        @pl.when(t >= 1)
        def _():
            pltpu.make_async_copy(msg, acc_sp.at[accidx[o]], ssem).wait()
            pltpu.make_async_copy(den_msg.at[pl.ds(0, CHUNK)],
                                  den_sp.at[denrv[o]], ssem).wait()

        pltpu.make_async_copy(kv_hbm.at[sgv[b]], kvb[b], gsem[b]).wait()
        pltpu.make_async_copy(q_hbm.at[dgv[b].at[pl.ds(0, CHUNK)]],
                              qb_[b], gsem[b]).wait()

        def idx_body(g, carry2):
            d16 = dgv[b][pl.ds(g * 16, 16)]
            dl = d16 - lo
            inh = (dl >= 0) & (dl < NHALF)
            accidx[b][pl.ds(g * 16, 16)] = jnp.where(
                inh, dl >> 2, ACC_DATA_ROWS)
            col4v[pl.ds(g * 16, 16)] = (dl & 3) * 32
            denrv[b][pl.ds(g * 16, 16)] = jnp.where(
                inh, dl >> 6, DEN_ROWS - 3)
            dencv[pl.ds(g * 16, 16)] = (dl & 63) * 2
            return carry2

        lax.fori_loop(0, CHUNK // 16, idx_body, 0)

        def edge_body(e, carry2):
            prev4 = prev4v[pl.ds(e, 16)][0]
            msg[e, pl.ds(prev4, 16)] = zero16
            msg[e, pl.ds(prev4 + 16, 16)] = zero16
            prevd = prevdv[pl.ds(e, 16)][0]
            den_msg[e, pl.ds(prevd, 16)] = zero16
            k0 = kvb[b][e, pl.ds(cb, 16)]
            k1 = kvb[b][e, pl.ds(cb + 16, 16)]
            v0 = kvb[b][e, pl.ds(cb + 32, 16)]
            v1 = kvb[b][e, pl.ds(cb + 48, 16)]
            q0 = qb_[b][e, pl.ds(qoff, 16)]
            q1 = qb_[b][e, pl.ds(qoff + 16, 16)]
            e0 = jnp.exp(_sum_splat(k0 * q0))
            e1 = jnp.exp(_sum_splat(k1 * q1))
            col4 = col4v[pl.ds(e, 16)][0]
            msg[e, pl.ds(col4, 16)] = v0 * e0
            msg[e, pl.ds(col4 + 16, 16)] = v1 * e1
            exd = jnp.where(lane == 0, e0, jnp.where(lane == 1, e1, 0.0))
            den_msg[e, pl.ds(dencv[pl.ds(e, 16)][0], 16)] = exd
            return carry2

        lax.fori_loop(0, CHUNK, edge_body, 0, unroll=4)

        def save_body(g, carry2):
            prev4v[pl.ds(g * 16, 16)] = col4v[pl.ds(g * 16, 16)]
            prevdv[pl.ds(g * 16, 16)] = dencv[pl.ds(g * 16, 16)]
            return carry2

        lax.fori_loop(0, CHUNK // 16, save_body, 0)

        pltpu.async_copy(msg, acc_sp.at[accidx[b]], ssem, add=True)
        pltpu.async_copy(den_msg.at[pl.ds(0, CHUNK)],
                         den_sp.at[denrv[b]], ssem, add=True)

        @pl.when(tp < PAIRS - 1)
        def _():
            load_and_fire(b, s + (t + 2) * NS)

    def pair_body(tp, carry):
        run_chunk(tp, 0)
        run_chunk(tp, 1)
        return carry

    lax.fori_loop(0, PAIRS, pair_body, 0)

    pltpu.make_async_copy(msg, acc_sp.at[accidx[1]], ssem).wait()
    pltpu.make_async_copy(den_msg.at[pl.ds(0, CHUNK)],
                          den_sp.at[denrv[1]], ssem).wait()
    plsc.subcore_barrier()

    pltpu.sync_copy(acc_sp.at[pl.ds(s * ACC_FLUSH, ACC_FLUSH)],
                    acc_out.at[c, s])

    @pl.when(s == 0)
    def _():
        pltpu.sync_copy(den_sp, den_out.at[c])


@functools.partial(jax.jit, static_argnums=0)
def _edge_pass(lo, kv, q2, sg, dg, zacc, zden):
    mesh = plsc.VectorSubcoreMesh(core_axis_name="c", subcore_axis_name="s",
                                  num_cores=NC, num_subcores=NS)
    return pl.kernel(
        functools.partial(_edge_body, lo),
        out_type=(
            jax.ShapeDtypeStruct((NC, NS, ACC_FLUSH, 128), jnp.float32),
            jax.ShapeDtypeStruct((NC, DEN_ROWS, 128), jnp.float32),
        ),
        mesh=mesh,
        scratch_types=[
            pltpu.VMEM_SHARED((ACC_ROWS, 128), jnp.float32),
            pltpu.VMEM_SHARED((DEN_ROWS, 128), jnp.float32),
            pltpu.VMEM((CHUNK,), jnp.int32),
            pltpu.VMEM((CHUNK,), jnp.int32),
            pltpu.VMEM((CHUNK + 16,), jnp.int32),
            pltpu.VMEM((CHUNK + 16,), jnp.int32),
            pltpu.VMEM((CHUNK, 128), jnp.float32),
            pltpu.VMEM((CHUNK, 128), jnp.float32),
            pltpu.VMEM((CHUNK, 128), jnp.float32),
            pltpu.VMEM((CHUNK, 128), jnp.float32),
            pltpu.VMEM((CHUNK,), jnp.int32),
            pltpu.VMEM((CHUNK,), jnp.int32),
            pltpu.VMEM((CHUNK,), jnp.int32),
            pltpu.VMEM((CHUNK,), jnp.int32),
            pltpu.VMEM((CHUNK + 16,), jnp.int32),
            pltpu.VMEM((CHUNK + 16,), jnp.int32),
            pltpu.VMEM((CHUNK + 16,), jnp.int32),
            pltpu.VMEM((CHUNK + 16,), jnp.int32),
            pltpu.VMEM((CHUNK, 128), jnp.float32),
            pltpu.VMEM((CHUNK + 1, 128), jnp.float32),
            pltpu.SemaphoreType.DMA,
            pltpu.SemaphoreType.DMA,
            pltpu.SemaphoreType.DMA,
        ],
        compiler_params=pltpu.CompilerParams(needs_layout_passes=False),
    )(kv, q2, sg, dg, zacc, zden)


def kernel(x_company, x_offshore_entity, x_person, edge_index_owns,
           edge_index_controls, edge_index_alias, edge_index_phoenix_successor,
           edge_index_issued_invoice_to, Win, b_in, Wkqv, b_kqv, Wk_rel,
           Wv_rel, p_rel, Wout, b_out, skip, Wc1, b_c1, Wc2, b_c2):
    NT = ['company', 'offshore_entity', 'person']
    xs = [x_company, x_offshore_entity, x_person]
    x = {}
    for i, nt in enumerate(NT):
        x[nt] = xs[i] @ Win[i] + b_in[i]

    ei = [edge_index_owns, edge_index_controls, edge_index_alias,
          edge_index_phoenix_successor, edge_index_issued_invoice_to]
    src_of = ['company', 'person', 'company', 'company', 'company']
    src_off = [0, 50000, 100000, 150000, 200000]
    npad = ETOTP - ETOT
    sg = jnp.concatenate(
        [ei[j][0] + src_off[j] for j in range(NET)]
        + [jnp.zeros((npad,), jnp.int32)])
    dg = jnp.concatenate([ei[j][1] for j in range(NET)]
                         + [jnp.full((npad,), N_COMPANY, jnp.int32)])
    zacc = jnp.zeros((ACC_FLUSH, 128), jnp.float32)
    zden = jnp.zeros((DEN_ROWS, 128), jnp.float32)

    inv_sqrt_d = 1.0 / math.sqrt(D)
    for l in range(NLAYER):
        kd, qd, vd = {}, {}, {}
        for i, nt in enumerate(NT):
            kqv = x[nt] @ Wkqv[l, i] + b_kqv[l, i]
            k_, q_, v_ = jnp.split(kqv, 3, axis=1)
            kd[nt] = k_.reshape(-1, H, D)
            qd[nt] = q_.reshape(-1, H, D)
            vd[nt] = v_.reshape(-1, H, D)
        q2 = jnp.pad(qd['company'].reshape(-1, F), ((0, 8), (0, 64)))
        ks_l, vs_l = [], []
        for j in range(NET):
            sname = src_of[j]
            idx = np.arange(H) * NET + j
            scale = (p_rel[l, j] * inv_sqrt_d)[None, :, None]
            ks_l.append(jnp.einsum('nhd,hde->nhe', kd[sname],
                                   Wk_rel[l][idx]) * scale)
            vs_l.append(jnp.einsum('nhd,hde->nhe', vd[sname], Wv_rel[l][idx]))
        k_all = jnp.concatenate(ks_l)
        v_all = jnp.concatenate(vs_l)
        kv = jnp.concatenate([
            k_all[:, 0:2].reshape(-1, 32), v_all[:, 0:2].reshape(-1, 32),
            k_all[:, 2:4].reshape(-1, 32), v_all[:, 2:4].reshape(-1, 32),
        ], axis=1)

        accs, dens = [], []
        for h in range(NPASS):
            acc_out, den_out = _edge_pass(h * NHALF, kv, q2, sg, dg,
                                          zacc, zden)
            acc_full = acc_out.reshape(NC, ACC_ROWS, 128)
            accs.append(jnp.concatenate(
                [acc_full[0, :ACC_DATA_ROWS].reshape(-1, 32),
                 acc_full[1, :ACC_DATA_ROWS].reshape(-1, 32)], axis=1))
            dens.append(jnp.concatenate(
                [den_out[0].reshape(-1, 2)[:NHALF],
                 den_out[1].reshape(-1, 2)[:NHALF]], axis=1))
        acc = jnp.concatenate(accs, axis=0)[:N_COMPANY]
        den = jnp.concatenate(dens, axis=0)[:N_COMPANY]
        o = (acc.reshape(-1, H, D) / (den[..., None] + 1e-16)).reshape(-1, F)
        a = jax.nn.gelu(o, approximate=False) @ Wout[l, 0] + b_out[l, 0]
        beta = jax.nn.sigmoid(skip[l, 0])
        x['company'] = jax.nn.elu(beta * a + (1.0 - beta) * x['company'])

    outs = []
    for i, nt in enumerate(NT):
        h1 = jax.nn.relu(x[nt] @ Wc1[i] + b_c1[i])
        outs.append((h1 @ Wc2[i] + b_c2[i])[:, 0])
    return tuple(outs)


# speedup vs baseline: 1.0290x; 1.0054x over previous
import functools
import math

import jax
import jax.numpy as jnp
import numpy as np
from jax import lax
from jax.experimental import pallas as pl
from jax.experimental.pallas import tpu as pltpu
from jax.experimental.pallas import tpu_sc as plsc

H = 4
F = 64
D = 16
NET = 5
NLAYER = 2
E = 160000
N_COMPANY = 50000
NHALF = 16672
NPASS = 3
ETOT = NET * E
CHUNK = 64
NCHUNK = 12512
ETOTP = NCHUNK * CHUNK
NC = 2
NS = 16
TRIPS = NCHUNK // NS
PAIRS = TRIPS // 2
ACC_DATA_ROWS = NHALF // 4
ACC_FLUSH = 264
ACC_ROWS = NS * ACC_FLUSH
DEN_ROWS = 264


def _edge_body(lo, kv_hbm, q_hbm, sg_hbm, dg_hbm, zacc_hbm, zden_hbm,
               acc_out, den_out,
               acc_sp, den_sp,
               sgv0, sgv1, dgv0, dgv1, kv0, kv1, q0b, q1b,
               accidx0, accidx1, denrv0, denrv1,
               col4v, prev4v, dencv, prevdv, msg, den_msg,
               gsem0, gsem1, ssem):
    c = lax.axis_index("c")
    s = lax.axis_index("s")

    sgv = (sgv0, sgv1)
    dgv = (dgv0, dgv1)
    kvb = (kv0, kv1)
    qb_ = (q0b, q1b)
    accidx = (accidx0, accidx1)
    denrv = (denrv0, denrv1)
    gsem = (gsem0, gsem1)

    pltpu.sync_copy(zacc_hbm, acc_sp.at[pl.ds(s * ACC_FLUSH, ACC_FLUSH)])

    @pl.when(s == 0)
    def _():
        pltpu.sync_copy(zden_hbm, den_sp)

    zero16 = jnp.zeros((16,), jnp.float32)
    zero16i = jnp.zeros((16,), jnp.int32)

    def zero_body(e, carry):
        for g in range(8):
            msg[e, pl.ds(g * 16, 16)] = zero16
        return carry

    lax.fori_loop(0, CHUNK, zero_body, 0)

    def zero_dben(e, carry):
        for g in range(8):
            den_msg[e, pl.ds(g * 16, 16)] = zero16
        return carry

    lax.fori_loop(0, CHUNK + 1, zero_dben, 0)

    def zero_idx_body(g, carry):
        prev4v[pl.ds(g * 16, 16)] = zero16i
        prevdv[pl.ds(g * 16, 16)] = zero16i
        return carry

    lax.fori_loop(0, (CHUNK + 16) // 16, zero_idx_body, 0)
    plsc.subcore_barrier()

    lane = lax.iota(jnp.int32, 16)
    perm = [lane ^ 8, lane ^ 4, lane ^ 2, lane ^ 1]

    def _sum_splat(x):
        for p in perm:
            x = x + x.at[p].get(mode="promise_in_bounds")
        return x

    cb = c * 64
    qoff = c * 32

    def load_and_fire(b, j):
        pltpu.sync_copy(sg_hbm.at[pl.ds(j * CHUNK, CHUNK)], sgv[b])
        pltpu.sync_copy(dg_hbm.at[pl.ds(j * CHUNK, CHUNK)],
                        dgv[b].at[pl.ds(0, CHUNK)])
        pltpu.async_copy(kv_hbm.at[sgv[b]], kvb[b], gsem[b])
        pltpu.async_copy(q_hbm.at[dgv[b].at[pl.ds(0, CHUNK)]], qb_[b],
                         gsem[b])

    for b in range(2):
        load_and_fire(b, s + b * NS)

    def run_chunk(tp, b):
        t = 2 * tp + b
        o = 1 - b

        @pl.when(t >= 1)
        def _():
            pltpu.make_async_copy(msg, acc_sp.at[accidx[o]], ssem).wait()
            pltpu.make_async_copy(den_msg.at[pl.ds(0, CHUNK)],
                                  den_sp.at[denrv[o]], ssem).wait()

        pltpu.make_async_copy(kv_hbm.at[sgv[b]], kvb[b], gsem[b]).wait()
        pltpu.make_async_copy(q_hbm.at[dgv[b].at[pl.ds(0, CHUNK)]],
                              qb_[b], gsem[b]).wait()

        def idx_body(g, carry2):
            d16 = dgv[b][pl.ds(g * 16, 16)]
            dl = d16 - lo
            inh = (dl >= 0) & (dl < NHALF)
            accidx[b][pl.ds(g * 16, 16)] = jnp.where(
                inh, dl >> 2, ACC_DATA_ROWS)
            col4v[pl.ds(g * 16, 16)] = (dl & 3) * 32
            denrv[b][pl.ds(g * 16, 16)] = jnp.where(
                inh, dl >> 6, DEN_ROWS - 3)
            dencv[pl.ds(g * 16, 16)] = (dl & 63) * 2
            return carry2

        lax.fori_loop(0, CHUNK // 16, idx_body, 0)

        def edge_body(e, carry2):
            prev4 = prev4v[pl.ds(e, 16)][0]
            msg[e, pl.ds(prev4, 16)] = zero16
            msg[e, pl.ds(prev4 + 16, 16)] = zero16
            prevd = prevdv[pl.ds(e, 16)][0]
            den_msg[e, pl.ds(prevd, 16)] = zero16
            k0 = kvb[b][e, pl.ds(cb, 16)]
            k1 = kvb[b][e, pl.ds(cb + 16, 16)]
            v0 = kvb[b][e, pl.ds(cb + 32, 16)]
            v1 = kvb[b][e, pl.ds(cb + 48, 16)]
            q0 = qb_[b][e, pl.ds(qoff, 16)]
            q1 = qb_[b][e, pl.ds(qoff + 16, 16)]
            e0 = jnp.exp(_sum_splat(k0 * q0))
            e1 = jnp.exp(_sum_splat(k1 * q1))
            col4 = col4v[pl.ds(e, 16)][0]
            msg[e, pl.ds(col4, 16)] = v0 * e0
            msg[e, pl.ds(col4 + 16, 16)] = v1 * e1
            exd = jnp.where(lane == 0, e0, jnp.where(lane == 1, e1, 0.0))
            den_msg[e, pl.ds(dencv[pl.ds(e, 16)][0], 16)] = exd
            return carry2

        lax.fori_loop(0, CHUNK, edge_body, 0, unroll=4)

        def save_body(g, carry2):
            prev4v[pl.ds(g * 16, 16)] = col4v[pl.ds(g * 16, 16)]
            prevdv[pl.ds(g * 16, 16)] = dencv[pl.ds(g * 16, 16)]
            return carry2

        lax.fori_loop(0, CHUNK // 16, save_body, 0)

        pltpu.async_copy(msg, acc_sp.at[accidx[b]], ssem, add=True)
        pltpu.async_copy(den_msg.at[pl.ds(0, CHUNK)],
                         den_sp.at[denrv[b]], ssem, add=True)

        @pl.when(tp < PAIRS - 1)
        def _():
            load_and_fire(b, s + (t + 2) * NS)

    def pair_body(tp, carry):
        run_chunk(tp, 0)
        run_chunk(tp, 1)
        return carry

    lax.fori_loop(0, PAIRS, pair_body, 0)

    pltpu.make_async_copy(msg, acc_sp.at[accidx[1]], ssem).wait()
    pltpu.make_async_copy(den_msg.at[pl.ds(0, CHUNK)],
                          den_sp.at[denrv[1]], ssem).wait()
    plsc.subcore_barrier()

    pltpu.sync_copy(acc_sp.at[pl.ds(s * ACC_FLUSH, ACC_FLUSH)],
                    acc_out.at[c, s])

    @pl.when(s == 0)
    def _():
        pltpu.sync_copy(den_sp, den_out.at[c])


@functools.partial(jax.jit, static_argnums=0)
def _edge_pass(lo, kv, q2, sg, dg, zacc, zden):
    mesh = plsc.VectorSubcoreMesh(core_axis_name="c", subcore_axis_name="s",
                                  num_cores=NC, num_subcores=NS)
    return pl.kernel(
        functools.partial(_edge_body, lo),
        out_type=(
            jax.ShapeDtypeStruct((NC, NS, ACC_FLUSH, 128), jnp.float32),
            jax.ShapeDtypeStruct((NC, DEN_ROWS, 128), jnp.float32),
        ),
        mesh=mesh,
        scratch_types=[
            pltpu.VMEM_SHARED((ACC_ROWS, 128), jnp.float32),
            pltpu.VMEM_SHARED((DEN_ROWS, 128), jnp.float32),
            pltpu.VMEM((CHUNK,), jnp.int32),
            pltpu.VMEM((CHUNK,), jnp.int32),
            pltpu.VMEM((CHUNK + 16,), jnp.int32),
            pltpu.VMEM((CHUNK + 16,), jnp.int32),
            pltpu.VMEM((CHUNK, 128), jnp.float32),
            pltpu.VMEM((CHUNK, 128), jnp.float32),
            pltpu.VMEM((CHUNK, 128), jnp.float32),
            pltpu.VMEM((CHUNK, 128), jnp.float32),
            pltpu.VMEM((CHUNK,), jnp.int32),
            pltpu.VMEM((CHUNK,), jnp.int32),
            pltpu.VMEM((CHUNK,), jnp.int32),
            pltpu.VMEM((CHUNK,), jnp.int32),
            pltpu.VMEM((CHUNK + 16,), jnp.int32),
            pltpu.VMEM((CHUNK + 16,), jnp.int32),
            pltpu.VMEM((CHUNK + 16,), jnp.int32),
            pltpu.VMEM((CHUNK + 16,), jnp.int32),
            pltpu.VMEM((CHUNK, 128), jnp.float32),
            pltpu.VMEM((CHUNK + 1, 128), jnp.float32),
            pltpu.SemaphoreType.DMA,
            pltpu.SemaphoreType.DMA,
            pltpu.SemaphoreType.DMA,
        ],
        compiler_params=pltpu.CompilerParams(needs_layout_passes=False),
    )(kv, q2, sg, dg, zacc, zden)


def _mm_body(x_ref, w_ref, b_ref, o_ref):
    o_ref[...] = x_ref[...] @ w_ref[...] + b_ref[...]


def _mm(x, w, b):
    n, k = x.shape
    m = w.shape[1]
    blk = 5000
    return pl.pallas_call(
        _mm_body,
        out_shape=jax.ShapeDtypeStruct((n, m), jnp.float32),
        grid=(n // blk,),
        in_specs=[
            pl.BlockSpec((blk, k), lambda i: (i, 0)),
            pl.BlockSpec((k, m), lambda i: (0, 0)),
            pl.BlockSpec((1, m), lambda i: (0, 0)),
        ],
        out_specs=pl.BlockSpec((blk, m), lambda i: (i, 0)),
    )(x, w, b.reshape(1, -1))


def kernel(x_company, x_offshore_entity, x_person, edge_index_owns,
           edge_index_controls, edge_index_alias, edge_index_phoenix_successor,
           edge_index_issued_invoice_to, Win, b_in, Wkqv, b_kqv, Wk_rel,
           Wv_rel, p_rel, Wout, b_out, skip, Wc1, b_c1, Wc2, b_c2):
    NT = ['company', 'offshore_entity', 'person']
    xs = [x_company, x_offshore_entity, x_person]
    x = {}
    for i, nt in enumerate(NT):
        x[nt] = _mm(xs[i], Win[i], b_in[i])

    ei = [edge_index_owns, edge_index_controls, edge_index_alias,
          edge_index_phoenix_successor, edge_index_issued_invoice_to]
    src_of = ['company', 'person', 'company', 'company', 'company']
    src_off = [0, 50000, 100000, 150000, 200000]
    npad = ETOTP - ETOT
    sg = jnp.concatenate(
        [ei[j][0] + src_off[j] for j in range(NET)]
        + [jnp.zeros((npad,), jnp.int32)])
    dg = jnp.concatenate([ei[j][1] for j in range(NET)]
                         + [jnp.full((npad,), N_COMPANY, jnp.int32)])
    zacc = jnp.zeros((ACC_FLUSH, 128), jnp.float32)
    zden = jnp.zeros((DEN_ROWS, 128), jnp.float32)

    inv_sqrt_d = 1.0 / math.sqrt(D)
    for l in range(NLAYER):
        kd, qd, vd = {}, {}, {}
        for i, nt in enumerate(NT):
            kqv = _mm(x[nt], Wkqv[l, i], b_kqv[l, i])
            k_, q_, v_ = jnp.split(kqv, 3, axis=1)
            kd[nt] = k_
            qd[nt] = q_
            vd[nt] = v_
        q2 = jnp.pad(qd['company'], ((0, 8), (0, 64)))
        zf = jnp.zeros((F,), jnp.float32)
        ks_l, vs_l = [], []
        for j in range(NET):
            sname = src_of[j]
            idx = np.arange(H) * NET + j
            wkb = jax.scipy.linalg.block_diag(
                *[Wk_rel[l, idx[hh]] * (p_rel[l, j, hh] * inv_sqrt_d)
                  for hh in range(H)])
            wvb = jax.scipy.linalg.block_diag(
                *[Wv_rel[l, idx[hh]] for hh in range(H)])
            ks_l.append(_mm(kd[sname], wkb, zf))
            vs_l.append(_mm(vd[sname], wvb, zf))
        k_all = jnp.concatenate(ks_l).reshape(-1, H, D)
        v_all = jnp.concatenate(vs_l).reshape(-1, H, D)
        kv = jnp.concatenate([
            k_all[:, 0:2].reshape(-1, 32), v_all[:, 0:2].reshape(-1, 32),
            k_all[:, 2:4].reshape(-1, 32), v_all[:, 2:4].reshape(-1, 32),
        ], axis=1)

        accs, dens = [], []
        for h in range(NPASS):
            acc_out, den_out = _edge_pass(h * NHALF, kv, q2, sg, dg,
                                          zacc, zden)
            acc_full = acc_out.reshape(NC, ACC_ROWS, 128)
            accs.append(jnp.concatenate(
                [acc_full[0, :ACC_DATA_ROWS].reshape(-1, 32),
                 acc_full[1, :ACC_DATA_ROWS].reshape(-1, 32)], axis=1))
            dens.append(jnp.concatenate(
                [den_out[0].reshape(-1, 2)[:NHALF],
                 den_out[1].reshape(-1, 2)[:NHALF]], axis=1))
        acc = jnp.concatenate(accs, axis=0)[:N_COMPANY]
        den = jnp.concatenate(dens, axis=0)[:N_COMPANY]
        o = (acc.reshape(-1, H, D) / (den[..., None] + 1e-16)).reshape(-1, F)
        a = _mm(jax.nn.gelu(o, approximate=False), Wout[l, 0], b_out[l, 0])
        beta = jax.nn.sigmoid(skip[l, 0])
        x['company'] = jax.nn.elu(beta * a + (1.0 - beta) * x['company'])

    outs = []
    for i, nt in enumerate(NT):
        h1 = jax.nn.relu(_mm(x[nt], Wc1[i], b_c1[i]))
        outs.append((h1 @ Wc2[i] + b_c2[i])[:, 0])
    return tuple(outs)


# speedup vs baseline: 1.4084x; 1.3686x over previous
import functools
import math

import jax
import jax.numpy as jnp
import numpy as np
from jax import lax
from jax.experimental import pallas as pl
from jax.experimental.pallas import tpu as pltpu
from jax.experimental.pallas import tpu_sc as plsc

H = 4
F = 64
D = 16
NET = 5
NLAYER = 2
E = 160000
N_COMPANY = 50000
NHALF = 16672
NPASS = 3
ETOT = NET * E
CHUNK = 64
NCHUNK = 12512
ETOTP = NCHUNK * CHUNK
NC = 2
NS = 16
TRIPS = NCHUNK // NS
PAIRS = TRIPS // 2
ACC_DATA_ROWS = NHALF // 4
ACC_FLUSH = 264
ACC_ROWS = NS * ACC_FLUSH
DEN_ROWS = 264


def _edge_body(lo, kv_hbm, q_hbm, sg_hbm, dg_hbm, zacc_hbm, zden_hbm,
               acc_out, den_out,
               acc_sp, den_sp,
               sgv0, sgv1, dgv0, dgv1, kv0, kv1, q0b, q1b,
               accidx0, accidx1, denrv0, denrv1,
               col4v, prev4v, dencv, prevdv, inhv, rowv, denrowv,
               msg, den_msg, gsem0, gsem1, ssem):
    c = lax.axis_index("c")
    s = lax.axis_index("s")

    sgv = (sgv0, sgv1)
    dgv = (dgv0, dgv1)
    kvb = (kv0, kv1)
    qb_ = (q0b, q1b)
    accidx = (accidx0, accidx1)
    denrv = (denrv0, denrv1)
    gsem = (gsem0, gsem1)

    pltpu.sync_copy(zacc_hbm, acc_sp.at[pl.ds(s * ACC_FLUSH, ACC_FLUSH)])

    @pl.when(s == 0)
    def _():
        pltpu.sync_copy(zden_hbm, den_sp)

    zero16 = jnp.zeros((16,), jnp.float32)
    zero16i = jnp.zeros((16,), jnp.int32)

    def zero_body(e, carry):
        for g in range(8):
            msg[e, pl.ds(g * 16, 16)] = zero16
        return carry

    lax.fori_loop(0, CHUNK, zero_body, 0)

    def zero_dben(e, carry):
        for g in range(8):
            den_msg[e, pl.ds(g * 16, 16)] = zero16
        return carry

    lax.fori_loop(0, CHUNK + 1, zero_dben, 0)

    def zero_idx_body(g, carry):
        prev4v[pl.ds(g * 16, 16)] = zero16i
        prevdv[pl.ds(g * 16, 16)] = zero16i
        return carry

    lax.fori_loop(0, (CHUNK + 16) // 16, zero_idx_body, 0)
    plsc.subcore_barrier()

    lane = lax.iota(jnp.int32, 16)
    perm = [lane ^ 8, lane ^ 4, lane ^ 2, lane ^ 1]

    def _sum_splat(x):
        for p in perm:
            x = x + x.at[p].get(mode="promise_in_bounds")
        return x

    cb = c * 64
    qoff = c * 32

    def load_and_fire(b, j):
        pltpu.sync_copy(sg_hbm.at[pl.ds(j * CHUNK, CHUNK)], sgv[b])
        pltpu.sync_copy(dg_hbm.at[pl.ds(j * CHUNK, CHUNK)],
                        dgv[b].at[pl.ds(0, CHUNK)])
        pltpu.async_copy(kv_hbm.at[sgv[b]], kvb[b], gsem[b])
        pltpu.async_copy(q_hbm.at[dgv[b].at[pl.ds(0, CHUNK)]], qb_[b],
                         gsem[b])

    for b in range(2):
        load_and_fire(b, s + b * NS)

    lane0 = lane == 0
    garb_acc = jnp.full((16,), ACC_DATA_ROWS, jnp.int32)
    garb_den = jnp.full((16,), DEN_ROWS - 3, jnp.int32)

    def run_chunk(tp, b):
        t = 2 * tp + b
        o = 1 - b

        @pl.when(t >= 1)
        def _():
            pltpu.make_async_copy(msg.at[pl.ds(0, 32)],
                                  acc_sp.at[accidx[o].at[0]], ssem).wait()
            pltpu.make_async_copy(den_msg.at[pl.ds(0, 32)],
                                  den_sp.at[denrv[o].at[0]], ssem).wait()

        pltpu.make_async_copy(kv_hbm.at[sgv[b]], kvb[b], gsem[b]).wait()
        pltpu.make_async_copy(q_hbm.at[dgv[b].at[pl.ds(0, CHUNK)]],
                              qb_[b], gsem[b]).wait()

        for r in range(2):
            accidx[b][r, pl.ds(0, 16)] = garb_acc
            accidx[b][r, pl.ds(16, 16)] = garb_acc
            denrv[b][r, pl.ds(0, 16)] = garb_den
            denrv[b][r, pl.ds(16, 16)] = garb_den

        def idx_body(g, carry2):
            d16 = dgv[b][pl.ds(g * 16, 16)]
            dl = d16 - lo
            inh = (dl >= 0) & (dl < NHALF)
            inhv[pl.ds(g * 16, 16)] = inh.astype(jnp.int32)
            rowv[pl.ds(g * 16, 16)] = dl >> 2
            col4v[pl.ds(g * 16, 16)] = (dl & 3) * 32
            denrowv[pl.ds(g * 16, 16)] = dl >> 6
            dencv[pl.ds(g * 16, 16)] = (dl & 63) * 2
            return carry2

        lax.fori_loop(0, CHUNK // 16, idx_body, 0)

        def edge_body(e, w):
            inh = inhv[pl.ds(e, 16)][0]

            @pl.when(inh > 0)
            def _():
                prev4 = prev4v[pl.ds(w, 16)][0]
                msg[w, pl.ds(prev4, 16)] = zero16
                msg[w, pl.ds(prev4 + 16, 16)] = zero16
                prevd = prevdv[pl.ds(w, 16)][0]
                den_msg[w, pl.ds(prevd, 16)] = zero16
                k0 = kvb[b][e, pl.ds(cb, 16)]
                k1 = kvb[b][e, pl.ds(cb + 16, 16)]
                v0 = kvb[b][e, pl.ds(cb + 32, 16)]
                v1 = kvb[b][e, pl.ds(cb + 48, 16)]
                q0 = qb_[b][e, pl.ds(qoff, 16)]
                q1 = qb_[b][e, pl.ds(qoff + 16, 16)]
                e0 = jnp.exp(_sum_splat(k0 * q0))
                e1 = jnp.exp(_sum_splat(k1 * q1))
                col4 = col4v[pl.ds(e, 16)][0]
                msg[w, pl.ds(col4, 16)] = v0 * e0
                msg[w, pl.ds(col4 + 16, 16)] = v1 * e1
                exd = jnp.where(lane == 0, e0,
                                jnp.where(lane == 1, e1, 0.0))
                dencol = dencv[pl.ds(e, 16)][0]
                den_msg[w, pl.ds(dencol, 16)] = exd
                plsc.store_scatter(prev4v, [jnp.full((16,), w, jnp.int32)],
                                   jnp.full((16,), col4, jnp.int32),
                                   mask=lane0)
                plsc.store_scatter(prevdv, [jnp.full((16,), w, jnp.int32)],
                                   jnp.full((16,), dencol, jnp.int32),
                                   mask=lane0)
                hi = jnp.full((16,), w >> 5, jnp.int32)
                lo16 = jnp.full((16,), w & 31, jnp.int32)
                plsc.store_scatter(
                    accidx[b], [hi, lo16],
                    jnp.full((16,), rowv[pl.ds(e, 16)][0], jnp.int32),
                    mask=lane0)
                plsc.store_scatter(
                    denrv[b], [hi, lo16],
                    jnp.full((16,), denrowv[pl.ds(e, 16)][0], jnp.int32),
                    mask=lane0)

            return w + inh

        cnt = lax.fori_loop(0, CHUNK, edge_body, 0)

        pltpu.async_copy(msg.at[pl.ds(0, 32)],
                         acc_sp.at[accidx[b].at[0]], ssem, add=True)
        pltpu.async_copy(den_msg.at[pl.ds(0, 32)],
                         den_sp.at[denrv[b].at[0]], ssem, add=True)

        @pl.when(cnt > 32)
        def _():
            pltpu.sync_copy(msg.at[pl.ds(32, 32)],
                            acc_sp.at[accidx[b].at[1]], add=True)
            pltpu.sync_copy(den_msg.at[pl.ds(32, 32)],
                            den_sp.at[denrv[b].at[1]], add=True)

        @pl.when(tp < PAIRS - 1)
        def _():
            load_and_fire(b, s + (t + 2) * NS)

    def pair_body(tp, carry):
        run_chunk(tp, 0)
        run_chunk(tp, 1)
        return carry

    lax.fori_loop(0, PAIRS, pair_body, 0)

    pltpu.make_async_copy(msg.at[pl.ds(0, 32)],
                          acc_sp.at[accidx[1].at[0]], ssem).wait()
    pltpu.make_async_copy(den_msg.at[pl.ds(0, 32)],
                          den_sp.at[denrv[1].at[0]], ssem).wait()
    plsc.subcore_barrier()

    pltpu.sync_copy(acc_sp.at[pl.ds(s * ACC_FLUSH, ACC_FLUSH)],
                    acc_out.at[c, s])

    @pl.when(s == 0)
    def _():
        pltpu.sync_copy(den_sp, den_out.at[c])


@functools.partial(jax.jit, static_argnums=0)
def _edge_pass(lo, kv, q2, sg, dg, zacc, zden):
    mesh = plsc.VectorSubcoreMesh(core_axis_name="c", subcore_axis_name="s",
                                  num_cores=NC, num_subcores=NS)
    return pl.kernel(
        functools.partial(_edge_body, lo),
        out_type=(
            jax.ShapeDtypeStruct((NC, NS, ACC_FLUSH, 128), jnp.float32),
            jax.ShapeDtypeStruct((NC, DEN_ROWS, 128), jnp.float32),
        ),
        mesh=mesh,
        scratch_types=[
            pltpu.VMEM_SHARED((ACC_ROWS, 128), jnp.float32),
            pltpu.VMEM_SHARED((DEN_ROWS, 128), jnp.float32),
            pltpu.VMEM((CHUNK,), jnp.int32),
            pltpu.VMEM((CHUNK,), jnp.int32),
            pltpu.VMEM((CHUNK + 16,), jnp.int32),
            pltpu.VMEM((CHUNK + 16,), jnp.int32),
            pltpu.VMEM((CHUNK, 128), jnp.float32),
            pltpu.VMEM((CHUNK, 128), jnp.float32),
            pltpu.VMEM((CHUNK, 128), jnp.float32),
            pltpu.VMEM((CHUNK, 128), jnp.float32),
            pltpu.VMEM((2, 32), jnp.int32),
            pltpu.VMEM((2, 32), jnp.int32),
            pltpu.VMEM((2, 32), jnp.int32),
            pltpu.VMEM((2, 32), jnp.int32),
            pltpu.VMEM((CHUNK + 16,), jnp.int32),
            pltpu.VMEM((CHUNK + 16,), jnp.int32),
            pltpu.VMEM((CHUNK + 16,), jnp.int32),
            pltpu.VMEM((CHUNK + 16,), jnp.int32),
            pltpu.VMEM((CHUNK + 16,), jnp.int32),
            pltpu.VMEM((CHUNK + 16,), jnp.int32),
            pltpu.VMEM((CHUNK + 16,), jnp.int32),
            pltpu.VMEM((CHUNK, 128), jnp.float32),
            pltpu.VMEM((CHUNK + 1, 128), jnp.float32),
            pltpu.SemaphoreType.DMA,
            pltpu.SemaphoreType.DMA,
            pltpu.SemaphoreType.DMA,
        ],
        compiler_params=pltpu.CompilerParams(needs_layout_passes=False),
    )(kv, q2, sg, dg, zacc, zden)


def _mm_body(x_ref, w_ref, b_ref, o_ref):
    o_ref[...] = x_ref[...] @ w_ref[...] + b_ref[...]


def _mm(x, w, b):
    n, k = x.shape
    m = w.shape[1]
    blk = 5000
    return pl.pallas_call(
        _mm_body,
        out_shape=jax.ShapeDtypeStruct((n, m), jnp.float32),
        grid=(n // blk,),
        in_specs=[
            pl.BlockSpec((blk, k), lambda i: (i, 0)),
            pl.BlockSpec((k, m), lambda i: (0, 0)),
            pl.BlockSpec((1, m), lambda i: (0, 0)),
        ],
        out_specs=pl.BlockSpec((blk, m), lambda i: (i, 0)),
    )(x, w, b.reshape(1, -1))


def kernel(x_company, x_offshore_entity, x_person, edge_index_owns,
           edge_index_controls, edge_index_alias, edge_index_phoenix_successor,
           edge_index_issued_invoice_to, Win, b_in, Wkqv, b_kqv, Wk_rel,
           Wv_rel, p_rel, Wout, b_out, skip, Wc1, b_c1, Wc2, b_c2):
    NT = ['company', 'offshore_entity', 'person']
    xs = [x_company, x_offshore_entity, x_person]
    x = {}
    for i, nt in enumerate(NT):
        x[nt] = _mm(xs[i], Win[i], b_in[i])

    ei = [edge_index_owns, edge_index_controls, edge_index_alias,
          edge_index_phoenix_successor, edge_index_issued_invoice_to]
    src_of = ['company', 'person', 'company', 'company', 'company']
    src_off = [0, 50000, 100000, 150000, 200000]
    npad = ETOTP - ETOT
    sg = jnp.concatenate(
        [ei[j][0] + src_off[j] for j in range(NET)]
        + [jnp.zeros((npad,), jnp.int32)])
    dg = jnp.concatenate([ei[j][1] for j in range(NET)]
                         + [jnp.full((npad,), N_COMPANY, jnp.int32)])
    zacc = jnp.zeros((ACC_FLUSH, 128), jnp.float32)
    zden = jnp.zeros((DEN_ROWS, 128), jnp.float32)

    inv_sqrt_d = 1.0 / math.sqrt(D)
    for l in range(NLAYER):
        kd, qd, vd = {}, {}, {}
        for i, nt in enumerate(NT):
            kqv = _mm(x[nt], Wkqv[l, i], b_kqv[l, i])
            k_, q_, v_ = jnp.split(kqv, 3, axis=1)
            kd[nt] = k_
            qd[nt] = q_
            vd[nt] = v_
        q2 = jnp.pad(qd['company'], ((0, 8), (0, 64)))
        zf = jnp.zeros((F,), jnp.float32)
        ks_l, vs_l = [], []
        for j in range(NET):
            sname = src_of[j]
            idx = np.arange(H) * NET + j
            wkb = jax.scipy.linalg.block_diag(
                *[Wk_rel[l, idx[hh]] * (p_rel[l, j, hh] * inv_sqrt_d)
                  for hh in range(H)])
            wvb = jax.scipy.linalg.block_diag(
                *[Wv_rel[l, idx[hh]] for hh in range(H)])
            ks_l.append(_mm(kd[sname], wkb, zf))
            vs_l.append(_mm(vd[sname], wvb, zf))
        k_all = jnp.concatenate(ks_l).reshape(-1, H, D)
        v_all = jnp.concatenate(vs_l).reshape(-1, H, D)
        kv = jnp.concatenate([
            k_all[:, 0:2].reshape(-1, 32), v_all[:, 0:2].reshape(-1, 32),
            k_all[:, 2:4].reshape(-1, 32), v_all[:, 2:4].reshape(-1, 32),
        ], axis=1)

        accs, dens = [], []
        for h in range(NPASS):
            acc_out, den_out = _edge_pass(h * NHALF, kv, q2, sg, dg,
                                          zacc, zden)
            acc_full = acc_out.reshape(NC, ACC_ROWS, 128)
            accs.append(jnp.concatenate(
                [acc_full[0, :ACC_DATA_ROWS].reshape(-1, 32),
                 acc_full[1, :ACC_DATA_ROWS].reshape(-1, 32)], axis=1))
            dens.append(jnp.concatenate(
                [den_out[0].reshape(-1, 2)[:NHALF],
                 den_out[1].reshape(-1, 2)[:NHALF]], axis=1))
        acc = jnp.concatenate(accs, axis=0)[:N_COMPANY]
        den = jnp.concatenate(dens, axis=0)[:N_COMPANY]
        o = (acc.reshape(-1, H, D) / (den[..., None] + 1e-16)).reshape(-1, F)
        a = _mm(jax.nn.gelu(o, approximate=False), Wout[l, 0], b_out[l, 0])
        beta = jax.nn.sigmoid(skip[l, 0])
        x['company'] = jax.nn.elu(beta * a + (1.0 - beta) * x['company'])

    outs = []
    for i, nt in enumerate(NT):
        h1 = jax.nn.relu(_mm(x[nt], Wc1[i], b_c1[i]))
        outs.append((h1 @ Wc2[i] + b_c2[i])[:, 0])
    return tuple(outs)


# speedup vs baseline: 1.4587x; 1.0357x over previous
import functools
import math

import jax
import jax.numpy as jnp
import numpy as np
from jax import lax
from jax.experimental import pallas as pl
from jax.experimental.pallas import tpu as pltpu
from jax.experimental.pallas import tpu_sc as plsc

H = 4
F = 64
D = 16
NET = 5
NLAYER = 2
E = 160000
N_COMPANY = 50000
NHALF = 16672
NPASS = 3
ETOT = NET * E
CHUNK = 64
NCHUNK = 12512
ETOTP = NCHUNK * CHUNK
NC = 2
NS = 16
TRIPS = NCHUNK // NS
PAIRS = TRIPS // 2
ACC_DATA_ROWS = NHALF // 4
ACC_FLUSH = 264
ACC_ROWS = NS * ACC_FLUSH
DEN_ROWS = 264


def _edge_body(lo, kv_hbm, q_hbm, sg_hbm, dg_hbm, zacc_hbm, zden_hbm,
               acc_out, den_out,
               acc_sp, den_sp,
               sgv0, sgv1, dgv0, dgv1, kv0, kv1, q0b, q1b,
               accidx0, accidx1, denrv0, denrv1,
               col4v, prev4v, dencv, prevdv, inhv, rowv, denrowv,
               msg, den_msg, gsem0, gsem1, ssem):
    c = lax.axis_index("c")
    s = lax.axis_index("s")

    sgv = (sgv0, sgv1)
    dgv = (dgv0, dgv1)
    kvb = (kv0, kv1)
    qb_ = (q0b, q1b)
    accidx = (accidx0, accidx1)
    denrv = (denrv0, denrv1)
    gsem = (gsem0, gsem1)

    pltpu.sync_copy(zacc_hbm, acc_sp.at[pl.ds(s * ACC_FLUSH, ACC_FLUSH)])

    @pl.when(s == 0)
    def _():
        pltpu.sync_copy(zden_hbm, den_sp)

    zero16 = jnp.zeros((16,), jnp.float32)
    zero16i = jnp.zeros((16,), jnp.int32)

    def zero_body(e, carry):
        for g in range(8):
            msg[e, pl.ds(g * 16, 16)] = zero16
        return carry

    lax.fori_loop(0, CHUNK, zero_body, 0)

    def zero_dben(e, carry):
        for g in range(8):
            den_msg[e, pl.ds(g * 16, 16)] = zero16
        return carry

    lax.fori_loop(0, CHUNK + 1, zero_dben, 0)

    def zero_idx_body(g, carry):
        prev4v[pl.ds(g * 16, 16)] = zero16i
        prevdv[pl.ds(g * 16, 16)] = zero16i
        return carry

    lax.fori_loop(0, (CHUNK + 16) // 16, zero_idx_body, 0)
    plsc.subcore_barrier()

    lane = lax.iota(jnp.int32, 16)
    perm = [lane ^ 8, lane ^ 4, lane ^ 2, lane ^ 1]

    def _sum_splat(x):
        for p in perm:
            x = x + x.at[p].get(mode="promise_in_bounds")
        return x

    cb = c * 64
    qoff = c * 32

    def load_and_fire(b, j):
        pltpu.sync_copy(sg_hbm.at[pl.ds(j * CHUNK, CHUNK)], sgv[b])
        pltpu.sync_copy(dg_hbm.at[pl.ds(j * CHUNK, CHUNK)],
                        dgv[b].at[pl.ds(0, CHUNK)])
        pltpu.async_copy(kv_hbm.at[sgv[b]], kvb[b], gsem[b])
        pltpu.async_copy(q_hbm.at[dgv[b].at[pl.ds(0, CHUNK)]], qb_[b],
                         gsem[b])

    for b in range(2):
        load_and_fire(b, s + b * NS)

    lane0 = lane == 0
    garb_acc = jnp.full((16,), ACC_DATA_ROWS, jnp.int32)
    garb_den = jnp.full((16,), DEN_ROWS - 3, jnp.int32)

    def run_chunk(tp, b):
        t = 2 * tp + b
        o = 1 - b

        @pl.when(t >= 1)
        def _():
            pltpu.make_async_copy(msg.at[pl.ds(0, 32)],
                                  acc_sp.at[accidx[o].at[0]], ssem).wait()
            pltpu.make_async_copy(den_msg.at[pl.ds(0, 32)],
                                  den_sp.at[denrv[o].at[0]], ssem).wait()

        pltpu.make_async_copy(kv_hbm.at[sgv[b]], kvb[b], gsem[b]).wait()
        pltpu.make_async_copy(q_hbm.at[dgv[b].at[pl.ds(0, CHUNK)]],
                              qb_[b], gsem[b]).wait()

        for r in range(2):
            accidx[b][r, pl.ds(0, 16)] = garb_acc
            accidx[b][r, pl.ds(16, 16)] = garb_acc
            denrv[b][r, pl.ds(0, 16)] = garb_den
            denrv[b][r, pl.ds(16, 16)] = garb_den

        def idx_body(g, carry2):
            d16 = dgv[b][pl.ds(g * 16, 16)]
            dl = d16 - lo
            inh = (dl >= 0) & (dl < NHALF)
            inhv[pl.ds(g * 16, 16)] = inh.astype(jnp.int32)
            rowv[pl.ds(g * 16, 16)] = dl >> 2
            col4v[pl.ds(g * 16, 16)] = (dl & 3) * 32
            denrowv[pl.ds(g * 16, 16)] = dl >> 6
            dencv[pl.ds(g * 16, 16)] = (dl & 63) * 2
            return carry2

        lax.fori_loop(0, CHUNK // 16, idx_body, 0)

        def edge_body(e, w):
            inh = inhv[pl.ds(e, 16)][0]

            @pl.when(inh > 0)
            def _():
                prev4 = prev4v[pl.ds(w, 16)][0]
                msg[w, pl.ds(prev4, 16)] = zero16
                msg[w, pl.ds(prev4 + 16, 16)] = zero16
                prevd = prevdv[pl.ds(w, 16)][0]
                den_msg[w, pl.ds(prevd, 16)] = zero16
                k0 = kvb[b][e, pl.ds(cb, 16)]
                k1 = kvb[b][e, pl.ds(cb + 16, 16)]
                v0 = kvb[b][e, pl.ds(cb + 32, 16)]
                v1 = kvb[b][e, pl.ds(cb + 48, 16)]
                q0 = qb_[b][e, pl.ds(qoff, 16)]
                q1 = qb_[b][e, pl.ds(qoff + 16, 16)]
                e0 = jnp.exp(_sum_splat(k0 * q0))
                e1 = jnp.exp(_sum_splat(k1 * q1))
                col4 = col4v[pl.ds(e, 16)][0]
                msg[w, pl.ds(col4, 16)] = v0 * e0
                msg[w, pl.ds(col4 + 16, 16)] = v1 * e1
                exd = jnp.where(lane == 0, e0,
                                jnp.where(lane == 1, e1, 0.0))
                dencol = dencv[pl.ds(e, 16)][0]
                den_msg[w, pl.ds(dencol, 16)] = exd
                plsc.store_scatter(prev4v, [jnp.full((16,), w, jnp.int32)],
                                   jnp.full((16,), col4, jnp.int32),
                                   mask=lane0)
                plsc.store_scatter(prevdv, [jnp.full((16,), w, jnp.int32)],
                                   jnp.full((16,), dencol, jnp.int32),
                                   mask=lane0)
                hi = jnp.full((16,), w >> 5, jnp.int32)
                lo16 = jnp.full((16,), w & 31, jnp.int32)
                plsc.store_scatter(
                    accidx[b], [hi, lo16],
                    jnp.full((16,), rowv[pl.ds(e, 16)][0], jnp.int32),
                    mask=lane0)
                plsc.store_scatter(
                    denrv[b], [hi, lo16],
                    jnp.full((16,), denrowv[pl.ds(e, 16)][0], jnp.int32),
                    mask=lane0)

            return w + inh

        cnt = lax.fori_loop(0, CHUNK, edge_body, 0, unroll=2)

        pltpu.async_copy(msg.at[pl.ds(0, 32)],
                         acc_sp.at[accidx[b].at[0]], ssem, add=True)
        pltpu.async_copy(den_msg.at[pl.ds(0, 32)],
                         den_sp.at[denrv[b].at[0]], ssem, add=True)

        @pl.when(cnt > 32)
        def _():
            pltpu.sync_copy(msg.at[pl.ds(32, 32)],
                            acc_sp.at[accidx[b].at[1]], add=True)
            pltpu.sync_copy(den_msg.at[pl.ds(32, 32)],
                            den_sp.at[denrv[b].at[1]], add=True)

        @pl.when(tp < PAIRS - 1)
        def _():
            load_and_fire(b, s + (t + 2) * NS)

    def pair_body(tp, carry):
        run_chunk(tp, 0)
        run_chunk(tp, 1)
        return carry

    lax.fori_loop(0, PAIRS, pair_body, 0)

    pltpu.make_async_copy(msg.at[pl.ds(0, 32)],
                          acc_sp.at[accidx[1].at[0]], ssem).wait()
    pltpu.make_async_copy(den_msg.at[pl.ds(0, 32)],
                          den_sp.at[denrv[1].at[0]], ssem).wait()
    plsc.subcore_barrier()

    pltpu.sync_copy(acc_sp.at[pl.ds(s * ACC_FLUSH, ACC_FLUSH)],
                    acc_out.at[c, s])

    @pl.when(s == 0)
    def _():
        pltpu.sync_copy(den_sp, den_out.at[c])


@functools.partial(jax.jit, static_argnums=0)
def _edge_pass(lo, kv, q2, sg, dg, zacc, zden):
    mesh = plsc.VectorSubcoreMesh(core_axis_name="c", subcore_axis_name="s",
                                  num_cores=NC, num_subcores=NS)
    return pl.kernel(
        functools.partial(_edge_body, lo),
        out_type=(
            jax.ShapeDtypeStruct((NC, NS, ACC_FLUSH, 128), jnp.float32),
            jax.ShapeDtypeStruct((NC, DEN_ROWS, 128), jnp.float32),
        ),
        mesh=mesh,
        scratch_types=[
            pltpu.VMEM_SHARED((ACC_ROWS, 128), jnp.float32),
            pltpu.VMEM_SHARED((DEN_ROWS, 128), jnp.float32),
            pltpu.VMEM((CHUNK,), jnp.int32),
            pltpu.VMEM((CHUNK,), jnp.int32),
            pltpu.VMEM((CHUNK + 16,), jnp.int32),
            pltpu.VMEM((CHUNK + 16,), jnp.int32),
            pltpu.VMEM((CHUNK, 128), jnp.float32),
            pltpu.VMEM((CHUNK, 128), jnp.float32),
            pltpu.VMEM((CHUNK, 128), jnp.float32),
            pltpu.VMEM((CHUNK, 128), jnp.float32),
            pltpu.VMEM((2, 32), jnp.int32),
            pltpu.VMEM((2, 32), jnp.int32),
            pltpu.VMEM((2, 32), jnp.int32),
            pltpu.VMEM((2, 32), jnp.int32),
            pltpu.VMEM((CHUNK + 16,), jnp.int32),
            pltpu.VMEM((CHUNK + 16,), jnp.int32),
            pltpu.VMEM((CHUNK + 16,), jnp.int32),
            pltpu.VMEM((CHUNK + 16,), jnp.int32),
            pltpu.VMEM((CHUNK + 16,), jnp.int32),
            pltpu.VMEM((CHUNK + 16,), jnp.int32),
            pltpu.VMEM((CHUNK + 16,), jnp.int32),
            pltpu.VMEM((CHUNK, 128), jnp.float32),
            pltpu.VMEM((CHUNK + 1, 128), jnp.float32),
            pltpu.SemaphoreType.DMA,
            pltpu.SemaphoreType.DMA,
            pltpu.SemaphoreType.DMA,
        ],
        compiler_params=pltpu.CompilerParams(needs_layout_passes=False),
    )(kv, q2, sg, dg, zacc, zden)


def _mm_body(x_ref, w_ref, b_ref, o_ref):
    o_ref[...] = x_ref[...] @ w_ref[...] + b_ref[...]


def _mm(x, w, b):
    n, k = x.shape
    m = w.shape[1]
    blk = 5000
    return pl.pallas_call(
        _mm_body,
        out_shape=jax.ShapeDtypeStruct((n, m), jnp.float32),
        grid=(n // blk,),
        in_specs=[
            pl.BlockSpec((blk, k), lambda i: (i, 0)),
            pl.BlockSpec((k, m), lambda i: (0, 0)),
            pl.BlockSpec((1, m), lambda i: (0, 0)),
        ],
        out_specs=pl.BlockSpec((blk, m), lambda i: (i, 0)),
    )(x, w, b.reshape(1, -1))


def kernel(x_company, x_offshore_entity, x_person, edge_index_owns,
           edge_index_controls, edge_index_alias, edge_index_phoenix_successor,
           edge_index_issued_invoice_to, Win, b_in, Wkqv, b_kqv, Wk_rel,
           Wv_rel, p_rel, Wout, b_out, skip, Wc1, b_c1, Wc2, b_c2):
    NT = ['company', 'offshore_entity', 'person']
    xs = [x_company, x_offshore_entity, x_person]
    x = {}
    for i, nt in enumerate(NT):
        x[nt] = _mm(xs[i], Win[i], b_in[i])

    ei = [edge_index_owns, edge_index_controls, edge_index_alias,
          edge_index_phoenix_successor, edge_index_issued_invoice_to]
    src_of = ['company', 'person', 'company', 'company', 'company']
    src_off = [0, 50000, 100000, 150000, 200000]
    npad = ETOTP - ETOT
    sg = jnp.concatenate(
        [ei[j][0] + src_off[j] for j in range(NET)]
        + [jnp.zeros((npad,), jnp.int32)])
    dg = jnp.concatenate([ei[j][1] for j in range(NET)]
                         + [jnp.full((npad,), N_COMPANY, jnp.int32)])
    zacc = jnp.zeros((ACC_FLUSH, 128), jnp.float32)
    zden = jnp.zeros((DEN_ROWS, 128), jnp.float32)

    inv_sqrt_d = 1.0 / math.sqrt(D)
    for l in range(NLAYER):
        kd, qd, vd = {}, {}, {}
        for i, nt in enumerate(NT):
            kqv = _mm(x[nt], Wkqv[l, i], b_kqv[l, i])
            k_, q_, v_ = jnp.split(kqv, 3, axis=1)
            kd[nt] = k_
            qd[nt] = q_
            vd[nt] = v_
        q2 = jnp.pad(qd['company'], ((0, 8), (0, 64)))
        zf = jnp.zeros((F,), jnp.float32)
        ks_l, vs_l = [], []
        for j in range(NET):
            sname = src_of[j]
            idx = np.arange(H) * NET + j
            wkb = jax.scipy.linalg.block_diag(
                *[Wk_rel[l, idx[hh]] * (p_rel[l, j, hh] * inv_sqrt_d)
                  for hh in range(H)])
            wvb = jax.scipy.linalg.block_diag(
                *[Wv_rel[l, idx[hh]] for hh in range(H)])
            ks_l.append(_mm(kd[sname], wkb, zf))
            vs_l.append(_mm(vd[sname], wvb, zf))
        k_all = jnp.concatenate(ks_l).reshape(-1, H, D)
        v_all = jnp.concatenate(vs_l).reshape(-1, H, D)
        kv = jnp.concatenate([
            k_all[:, 0:2].reshape(-1, 32), v_all[:, 0:2].reshape(-1, 32),
            k_all[:, 2:4].reshape(-1, 32), v_all[:, 2:4].reshape(-1, 32),
        ], axis=1)

        accs, dens = [], []
        for h in range(NPASS):
            acc_out, den_out = _edge_pass(h * NHALF, kv, q2, sg, dg,
                                          zacc, zden)
            acc_full = acc_out.reshape(NC, ACC_ROWS, 128)
            accs.append(jnp.concatenate(
                [acc_full[0, :ACC_DATA_ROWS].reshape(-1, 32),
                 acc_full[1, :ACC_DATA_ROWS].reshape(-1, 32)], axis=1))
            dens.append(jnp.concatenate(
                [den_out[0].reshape(-1, 2)[:NHALF],
                 den_out[1].reshape(-1, 2)[:NHALF]], axis=1))
        acc = jnp.concatenate(accs, axis=0)[:N_COMPANY]
        den = jnp.concatenate(dens, axis=0)[:N_COMPANY]
        o = (acc.reshape(-1, H, D) / (den[..., None] + 1e-16)).reshape(-1, F)
        a = _mm(jax.nn.gelu(o, approximate=False), Wout[l, 0], b_out[l, 0])
        beta = jax.nn.sigmoid(skip[l, 0])
        x['company'] = jax.nn.elu(beta * a + (1.0 - beta) * x['company'])

    outs = []
    for i, nt in enumerate(NT):
        h1 = jax.nn.relu(_mm(x[nt], Wc1[i], b_c1[i]))
        outs.append((h1 @ Wc2[i] + b_c2[i])[:, 0])
    return tuple(outs)


# speedup vs baseline: 1.4785x; 1.0136x over previous
import functools
import math

import jax
import jax.numpy as jnp
import numpy as np
from jax import lax
from jax.experimental import pallas as pl
from jax.experimental.pallas import tpu as pltpu
from jax.experimental.pallas import tpu_sc as plsc

H = 4
F = 64
D = 16
NET = 5
NLAYER = 2
E = 160000
N_COMPANY = 50000
NHALF = 16672
NPASS = 3
ETOT = NET * E
CHUNK = 64
NCHUNK = 12512
ETOTP = NCHUNK * CHUNK
NC = 2
NS = 16
TRIPS = NCHUNK // NS
PAIRS = TRIPS // 2
ACC_DATA_ROWS = NHALF // 4
ACC_FLUSH = 264
ACC_ROWS = NS * ACC_FLUSH
DEN_ROWS = 264


def _edge_body(lo, kv_hbm, q_hbm, sg_hbm, dg_hbm, zacc_hbm, zden_hbm,
               acc_out, den_out,
               acc_sp, den_sp,
               sgv0, sgv1, dgv0, dgv1, kv0, kv1, q0b, q1b,
               accidx0, accidx1, denrv0, denrv1,
               col4v, prev4v, dencv, prevdv, inhv, rowv, denrowv,
               msg, den_msg, gsem0, gsem1, ssem):
    c = lax.axis_index("c")
    s = lax.axis_index("s")

    sgv = (sgv0, sgv1)
    dgv = (dgv0, dgv1)
    kvb = (kv0, kv1)
    qb_ = (q0b, q1b)
    accidx = (accidx0, accidx1)
    denrv = (denrv0, denrv1)
    gsem = (gsem0, gsem1)

    pltpu.sync_copy(zacc_hbm, acc_sp.at[pl.ds(s * ACC_FLUSH, ACC_FLUSH)])

    @pl.when(s == 0)
    def _():
        pltpu.sync_copy(zden_hbm, den_sp)

    zero16 = jnp.zeros((16,), jnp.float32)
    zero16i = jnp.zeros((16,), jnp.int32)

    def zero_body(e, carry):
        for g in range(8):
            msg[e, pl.ds(g * 16, 16)] = zero16
        return carry

    lax.fori_loop(0, CHUNK, zero_body, 0)

    def zero_dben(e, carry):
        for g in range(8):
            den_msg[e, pl.ds(g * 16, 16)] = zero16
        return carry

    lax.fori_loop(0, CHUNK + 1, zero_dben, 0)

    def zero_idx_body(g, carry):
        prev4v[pl.ds(g * 16, 16)] = zero16i
        prevdv[pl.ds(g * 16, 16)] = zero16i
        return carry

    lax.fori_loop(0, (CHUNK + 16) // 16, zero_idx_body, 0)
    plsc.subcore_barrier()

    lane = lax.iota(jnp.int32, 16)
    perm = [lane ^ 8, lane ^ 4, lane ^ 2, lane ^ 1]

    def _sum_splat(x):
        for p in perm:
            x = x + x.at[p].get(mode="promise_in_bounds")
        return x

    cb = c * 64
    qoff = c * 32

    def load_and_fire(b, j):
        pltpu.sync_copy(sg_hbm.at[pl.ds(j * CHUNK, CHUNK)], sgv[b])
        pltpu.sync_copy(dg_hbm.at[pl.ds(j * CHUNK, CHUNK)],
                        dgv[b].at[pl.ds(0, CHUNK)])
        pltpu.async_copy(kv_hbm.at[sgv[b]], kvb[b], gsem[b])
        pltpu.async_copy(q_hbm.at[dgv[b].at[pl.ds(0, CHUNK)]], qb_[b],
                         gsem[b])

    for b in range(2):
        load_and_fire(b, s + b * NS)

    lane0 = lane == 0
    garb_acc = jnp.full((16,), ACC_DATA_ROWS, jnp.int32)
    garb_den = jnp.full((16,), DEN_ROWS - 3, jnp.int32)

    def run_chunk(tp, b):
        t = 2 * tp + b
        o = 1 - b

        @pl.when(t >= 1)
        def _():
            pltpu.make_async_copy(msg.at[pl.ds(0, 32)],
                                  acc_sp.at[accidx[o].at[0]], ssem).wait()
            pltpu.make_async_copy(den_msg.at[pl.ds(0, 32)],
                                  den_sp.at[denrv[o].at[0]], ssem).wait()

        pltpu.make_async_copy(kv_hbm.at[sgv[b]], kvb[b], gsem[b]).wait()
        pltpu.make_async_copy(q_hbm.at[dgv[b].at[pl.ds(0, CHUNK)]],
                              qb_[b], gsem[b]).wait()

        for r in range(2):
            accidx[b][r, pl.ds(0, 16)] = garb_acc
            accidx[b][r, pl.ds(16, 16)] = garb_acc
            denrv[b][r, pl.ds(0, 16)] = garb_den
            denrv[b][r, pl.ds(16, 16)] = garb_den

        def idx_body(g, carry2):
            d16 = dgv[b][pl.ds(g * 16, 16)]
            dl = d16 - lo
            inh = (dl >= 0) & (dl < NHALF)
            inhv[pl.ds(g * 16, 16)] = inh.astype(jnp.int32)
            rowv[pl.ds(g * 16, 16)] = dl >> 2
            col4v[pl.ds(g * 16, 16)] = (dl & 3) * 32
            denrowv[pl.ds(g * 16, 16)] = dl >> 6
            dencv[pl.ds(g * 16, 16)] = (dl & 63) * 2
            return carry2

        lax.fori_loop(0, CHUNK // 16, idx_body, 0)

        def edge_body(e, w):
            inh = inhv[pl.ds(e, 16)][0]

            @pl.when(inh > 0)
            def _():
                prev4 = prev4v[pl.ds(w, 16)][0]
                msg[w, pl.ds(prev4, 16)] = zero16
                msg[w, pl.ds(prev4 + 16, 16)] = zero16
                prevd = prevdv[pl.ds(w, 16)][0]
                den_msg[w, pl.ds(prevd, 16)] = zero16
                k0 = kvb[b][e, pl.ds(cb, 16)]
                k1 = kvb[b][e, pl.ds(cb + 16, 16)]
                v0 = kvb[b][e, pl.ds(cb + 32, 16)]
                v1 = kvb[b][e, pl.ds(cb + 48, 16)]
                q0 = qb_[b][e, pl.ds(qoff, 16)]
                q1 = qb_[b][e, pl.ds(qoff + 16, 16)]
                e0 = jnp.exp(_sum_splat(k0 * q0))
                e1 = jnp.exp(_sum_splat(k1 * q1))
                col4 = col4v[pl.ds(e, 16)][0]
                msg[w, pl.ds(col4, 16)] = v0 * e0
                msg[w, pl.ds(col4 + 16, 16)] = v1 * e1
                exd = jnp.where(lane == 0, e0,
                                jnp.where(lane == 1, e1, 0.0))
                dencol = dencv[pl.ds(e, 16)][0]
                den_msg[w, pl.ds(dencol, 16)] = exd
                plsc.store_scatter(prev4v, [jnp.full((16,), w, jnp.int32)],
                                   jnp.full((16,), col4, jnp.int32),
                                   mask=lane0)
                plsc.store_scatter(prevdv, [jnp.full((16,), w, jnp.int32)],
                                   jnp.full((16,), dencol, jnp.int32),
                                   mask=lane0)
                hi = jnp.full((16,), w >> 5, jnp.int32)
                lo16 = jnp.full((16,), w & 31, jnp.int32)
                plsc.store_scatter(
                    accidx[b], [hi, lo16],
                    jnp.full((16,), rowv[pl.ds(e, 16)][0], jnp.int32),
                    mask=lane0)
                plsc.store_scatter(
                    denrv[b], [hi, lo16],
                    jnp.full((16,), denrowv[pl.ds(e, 16)][0], jnp.int32),
                    mask=lane0)

            return w + inh

        cnt = lax.fori_loop(0, CHUNK, edge_body, 0, unroll=4)

        pltpu.async_copy(msg.at[pl.ds(0, 32)],
                         acc_sp.at[accidx[b].at[0]], ssem, add=True)
        pltpu.async_copy(den_msg.at[pl.ds(0, 32)],
                         den_sp.at[denrv[b].at[0]], ssem, add=True)

        @pl.when(cnt > 32)
        def _():
            pltpu.sync_copy(msg.at[pl.ds(32, 32)],
                            acc_sp.at[accidx[b].at[1]], add=True)
            pltpu.sync_copy(den_msg.at[pl.ds(32, 32)],
                            den_sp.at[denrv[b].at[1]], add=True)

        @pl.when(tp < PAIRS - 1)
        def _():
            load_and_fire(b, s + (t + 2) * NS)

    def pair_body(tp, carry):
        run_chunk(tp, 0)
        run_chunk(tp, 1)
        return carry

    lax.fori_loop(0, PAIRS, pair_body, 0)

    pltpu.make_async_copy(msg.at[pl.ds(0, 32)],
                          acc_sp.at[accidx[1].at[0]], ssem).wait()
    pltpu.make_async_copy(den_msg.at[pl.ds(0, 32)],
                          den_sp.at[denrv[1].at[0]], ssem).wait()
    plsc.subcore_barrier()

    pltpu.sync_copy(acc_sp.at[pl.ds(s * ACC_FLUSH, ACC_FLUSH)],
                    acc_out.at[c, s])

    @pl.when(s == 0)
    def _():
        pltpu.sync_copy(den_sp, den_out.at[c])


@functools.partial(jax.jit, static_argnums=0)
def _edge_pass(lo, kv, q2, sg, dg, zacc, zden):
    mesh = plsc.VectorSubcoreMesh(core_axis_name="c", subcore_axis_name="s",
                                  num_cores=NC, num_subcores=NS)
    return pl.kernel(
        functools.partial(_edge_body, lo),
        out_type=(
            jax.ShapeDtypeStruct((NC, NS, ACC_FLUSH, 128), jnp.float32),
            jax.ShapeDtypeStruct((NC, DEN_ROWS, 128), jnp.float32),
        ),
        mesh=mesh,
        scratch_types=[
            pltpu.VMEM_SHARED((ACC_ROWS, 128), jnp.float32),
            pltpu.VMEM_SHARED((DEN_ROWS, 128), jnp.float32),
            pltpu.VMEM((CHUNK,), jnp.int32),
            pltpu.VMEM((CHUNK,), jnp.int32),
            pltpu.VMEM((CHUNK + 16,), jnp.int32),
            pltpu.VMEM((CHUNK + 16,), jnp.int32),
            pltpu.VMEM((CHUNK, 128), jnp.float32),
            pltpu.VMEM((CHUNK, 128), jnp.float32),
            pltpu.VMEM((CHUNK, 128), jnp.float32),
            pltpu.VMEM((CHUNK, 128), jnp.float32),
            pltpu.VMEM((2, 32), jnp.int32),
            pltpu.VMEM((2, 32), jnp.int32),
            pltpu.VMEM((2, 32), jnp.int32),
            pltpu.VMEM((2, 32), jnp.int32),
            pltpu.VMEM((CHUNK + 16,), jnp.int32),
            pltpu.VMEM((CHUNK + 16,), jnp.int32),
            pltpu.VMEM((CHUNK + 16,), jnp.int32),
            pltpu.VMEM((CHUNK + 16,), jnp.int32),
            pltpu.VMEM((CHUNK + 16,), jnp.int32),
            pltpu.VMEM((CHUNK + 16,), jnp.int32),
            pltpu.VMEM((CHUNK + 16,), jnp.int32),
            pltpu.VMEM((CHUNK, 128), jnp.float32),
            pltpu.VMEM((CHUNK + 1, 128), jnp.float32),
            pltpu.SemaphoreType.DMA,
            pltpu.SemaphoreType.DMA,
            pltpu.SemaphoreType.DMA,
        ],
        compiler_params=pltpu.CompilerParams(needs_layout_passes=False),
    )(kv, q2, sg, dg, zacc, zden)


def _mm_body(x_ref, w_ref, b_ref, o_ref):
    o_ref[...] = x_ref[...] @ w_ref[...] + b_ref[...]


def _mm(x, w, b):
    n, k = x.shape
    m = w.shape[1]
    blk = 5000
    return pl.pallas_call(
        _mm_body,
        out_shape=jax.ShapeDtypeStruct((n, m), jnp.float32),
        grid=(n // blk,),
        in_specs=[
            pl.BlockSpec((blk, k), lambda i: (i, 0)),
            pl.BlockSpec((k, m), lambda i: (0, 0)),
            pl.BlockSpec((1, m), lambda i: (0, 0)),
        ],
        out_specs=pl.BlockSpec((blk, m), lambda i: (i, 0)),
    )(x, w, b.reshape(1, -1))


def kernel(x_company, x_offshore_entity, x_person, edge_index_owns,
           edge_index_controls, edge_index_alias, edge_index_phoenix_successor,
           edge_index_issued_invoice_to, Win, b_in, Wkqv, b_kqv, Wk_rel,
           Wv_rel, p_rel, Wout, b_out, skip, Wc1, b_c1, Wc2, b_c2):
    NT = ['company', 'offshore_entity', 'person']
    xs = [x_company, x_offshore_entity, x_person]
    x = {}
    for i, nt in enumerate(NT):
        x[nt] = _mm(xs[i], Win[i], b_in[i])

    ei = [edge_index_owns, edge_index_controls, edge_index_alias,
          edge_index_phoenix_successor, edge_index_issued_invoice_to]
    src_of = ['company', 'person', 'company', 'company', 'company']
    src_off = [0, 50000, 100000, 150000, 200000]
    npad = ETOTP - ETOT
    sg = jnp.concatenate(
        [ei[j][0] + src_off[j] for j in range(NET)]
        + [jnp.zeros((npad,), jnp.int32)])
    dg = jnp.concatenate([ei[j][1] for j in range(NET)]
                         + [jnp.full((npad,), N_COMPANY, jnp.int32)])
    zacc = jnp.zeros((ACC_FLUSH, 128), jnp.float32)
    zden = jnp.zeros((DEN_ROWS, 128), jnp.float32)

    inv_sqrt_d = 1.0 / math.sqrt(D)
    for l in range(NLAYER):
        kd, qd, vd = {}, {}, {}
        for i, nt in enumerate(NT):
            kqv = _mm(x[nt], Wkqv[l, i], b_kqv[l, i])
            k_, q_, v_ = jnp.split(kqv, 3, axis=1)
            kd[nt] = k_
            qd[nt] = q_
            vd[nt] = v_
        q2 = jnp.pad(qd['company'], ((0, 8), (0, 64)))
        zf = jnp.zeros((F,), jnp.float32)
        ks_l, vs_l = [], []
        for j in range(NET):
            sname = src_of[j]
            idx = np.arange(H) * NET + j
            wkb = jax.scipy.linalg.block_diag(
                *[Wk_rel[l, idx[hh]] * (p_rel[l, j, hh] * inv_sqrt_d)
                  for hh in range(H)])
            wvb = jax.scipy.linalg.block_diag(
                *[Wv_rel[l, idx[hh]] for hh in range(H)])
            ks_l.append(_mm(kd[sname], wkb, zf))
            vs_l.append(_mm(vd[sname], wvb, zf))
        k_all = jnp.concatenate(ks_l).reshape(-1, H, D)
        v_all = jnp.concatenate(vs_l).reshape(-1, H, D)
        kv = jnp.concatenate([
            k_all[:, 0:2].reshape(-1, 32), v_all[:, 0:2].reshape(-1, 32),
            k_all[:, 2:4].reshape(-1, 32), v_all[:, 2:4].reshape(-1, 32),
        ], axis=1)

        accs, dens = [], []
        for h in range(NPASS):
            acc_out, den_out = _edge_pass(h * NHALF, kv, q2, sg, dg,
                                          zacc, zden)
            acc_full = acc_out.reshape(NC, ACC_ROWS, 128)
            accs.append(jnp.concatenate(
                [acc_full[0, :ACC_DATA_ROWS].reshape(-1, 32),
                 acc_full[1, :ACC_DATA_ROWS].reshape(-1, 32)], axis=1))
            dens.append(jnp.concatenate(
                [den_out[0].reshape(-1, 2)[:NHALF],
                 den_out[1].reshape(-1, 2)[:NHALF]], axis=1))
        acc = jnp.concatenate(accs, axis=0)[:N_COMPANY]
        den = jnp.concatenate(dens, axis=0)[:N_COMPANY]
        o = (acc.reshape(-1, H, D) / (den[..., None] + 1e-16)).reshape(-1, F)
        a = _mm(jax.nn.gelu(o, approximate=False), Wout[l, 0], b_out[l, 0])
        beta = jax.nn.sigmoid(skip[l, 0])
        x['company'] = jax.nn.elu(beta * a + (1.0 - beta) * x['company'])

    outs = []
    for i, nt in enumerate(NT):
        h1 = jax.nn.relu(_mm(x[nt], Wc1[i], b_c1[i]))
        outs.append((h1 @ Wc2[i] + b_c2[i])[:, 0])
    return tuple(outs)
